# Initial kernel scaffold; baseline (speedup 1.0000x reference)
import jax
import jax.numpy as jnp
from jax.experimental import pallas as pl


def kernel(x, edge_index, batch, W1, b1, lin_W, lin_b, att_W, att_b, le1_W, le1_b, le2_W, le3_W, le3_b, W2, b2):
    raise NotImplementedError("write your pallas kernel here")


# speedup vs baseline: 1.3212x; 1.3212x over previous
import jax
import jax.numpy as jnp
import numpy as np
from jax.experimental import pallas as pl

N = 4096
E = 65536
K = 2048
D = 128


def _mm_kernel(a_ref, b_ref, o_ref):
    o_ref[...] = jnp.dot(a_ref[...], b_ref[...],
                         preferred_element_type=jnp.float32)


def _matmul(a, b):
    return pl.pallas_call(
        _mm_kernel,
        out_shape=jax.ShapeDtypeStruct((a.shape[0], b.shape[1]), jnp.float32),
    )(a, b)


def _mm_acc_kernel(a_ref, b_ref, o_ref):
    @pl.when(pl.program_id(2) == 0)
    def _():
        o_ref[...] = jnp.zeros_like(o_ref)

    o_ref[...] += jnp.dot(a_ref[...], b_ref[...],
                          preferred_element_type=jnp.float32)


def _matmul_big(a, b, bm=512, bn=1024, bk=1024):
    M, Kd = a.shape
    _, Nd = b.shape
    return pl.pallas_call(
        _mm_acc_kernel,
        grid=(M // bm, Nd // bn, Kd // bk),
        in_specs=[pl.BlockSpec((bm, bk), lambda i, j, k: (i, k)),
                  pl.BlockSpec((bk, bn), lambda i, j, k: (k, j))],
        out_specs=pl.BlockSpec((bm, bn), lambda i, j, k: (i, j)),
        out_shape=jax.ShapeDtypeStruct((M, Nd), jnp.float32),
    )(a, b)


def _mm_at_acc_kernel(a_ref, b_ref, o_ref):
    @pl.when(pl.program_id(2) == 0)
    def _():
        o_ref[...] = jnp.zeros_like(o_ref)

    o_ref[...] += jax.lax.dot_general(
        a_ref[...], b_ref[...], (((0,), (0,)), ((), ())),
        preferred_element_type=jnp.float32)


def _matmul_at(a, b, bm=512, bn=1024, bk=1024):
    Kd, M = a.shape
    _, Nd = b.shape
    return pl.pallas_call(
        _mm_at_acc_kernel,
        grid=(M // bm, Nd // bn, Kd // bk),
        in_specs=[pl.BlockSpec((bk, bm), lambda i, j, k: (k, i)),
                  pl.BlockSpec((bk, bn), lambda i, j, k: (k, j))],
        out_specs=pl.BlockSpec((bm, bn), lambda i, j, k: (i, j)),
        out_shape=jax.ShapeDtypeStruct((M, Nd), jnp.float32),
    )(a, b)


def kernel(x, edge_index, batch, W1, b1, lin_W, lin_b, att_W, att_b,
           le1_W, le1_b, le2_W, le3_W, le3_b, W2, b2):
    row, col = edge_index[0], edge_index[1]
    f32 = jnp.float32

    ones = jnp.ones((E,), f32)
    deg = jax.ops.segment_sum(ones, col, num_segments=N) + 1.0
    dinv = 1.0 / jnp.sqrt(deg)
    xw1 = _matmul(x, W1)
    Y = dinv[:, None] * xw1
    agg1 = jax.ops.segment_sum(Y[row], col, num_segments=N) + Y
    h = jax.nn.relu(dinv[:, None] * agg1 + b1)

    x_q = jnp.maximum(jax.ops.segment_max(h[row], col, num_segments=N), h)
    u = att_W[0, :D]
    v = att_W[0, D:]
    g = lin_W.T @ u
    c0 = jnp.dot(u, lin_b) + att_b[0]
    q = x_q @ g + c0
    p = h @ v
    pm = jnp.max(p)
    shift = jax.nn.leaky_relu(q + pm, 0.2)

    s_e = jax.nn.leaky_relu(q[col] + p[row], 0.2)
    sexp_e = jnp.exp(s_e - shift[col])
    sexp_self = jnp.exp(jax.nn.leaky_relu(q + p, 0.2) - shift)
    ssum = jax.ops.segment_sum(sexp_e, col, num_segments=N) + sexp_self
    den = ssum + 1e-16
    xnew_num = (jax.ops.segment_sum(sexp_e[:, None] * h[row], col,
                                    num_segments=N)
                + sexp_self[:, None] * h)
    x_new = xnew_num / den[:, None]

    a_s = x_new @ le1_W[0] + le1_b[0]
    bb_s = x_new @ le2_W[0]
    t_s = x_new @ le3_W[0] + le3_b[0]
    asum = jax.ops.segment_sum(a_s[row], col, num_segments=N) + a_s
    fitness = jax.nn.sigmoid(asum - deg * bb_s + t_s)

    _, perm = jax.lax.top_k(fitness, K)
    m = jnp.zeros((N,), f32).at[perm].set(1.0)

    Zc = jnp.zeros((N, N), f32).at[row, col].add(1.0)
    Zp = ((Zc + jnp.eye(N, dtype=f32)) > 0).astype(jnp.bfloat16)
    mb = m.astype(jnp.bfloat16)
    Zm = Zp * mb[None, :]
    U = _matmul_big(Zp, Zm)
    Up = (U > 0).astype(jnp.bfloat16)
    V = _matmul_at(Zm, Up)

    offdiag = 1.0 - jnp.eye(N, dtype=f32)
    Bm = (V > 0).astype(f32) * offdiag * (m[:, None] * m[None, :]) \
        + jnp.diag(m)
    deg2 = Bm.sum(axis=0)
    dinv2 = jnp.where(deg2 > 0, 1.0 / jnp.sqrt(deg2), 0.0) * m
    r = Bm @ dinv2
    w = dinv2 * r
    coeff = m * fitness * w * (1.0 / K)
    y = coeff @ x_new
    out = y @ W2 + b2
    return out[None, :]


# speedup vs baseline: 1.3879x; 1.0504x over previous
import jax
import jax.numpy as jnp
import numpy as np
from jax.experimental import pallas as pl
from jax.experimental.pallas import tpu as pltpu

N = 4096
E = 65536
K = 2048
D = 128


def _zp_kernel(zc_ref, zp_ref):
    i = pl.program_id(0)
    bm = zp_ref.shape[0]
    rowi = i * bm + jax.lax.broadcasted_iota(jnp.int32, zp_ref.shape, 0)
    coli = jax.lax.broadcasted_iota(jnp.int32, zp_ref.shape, 1)
    zp_ref[...] = ((zc_ref[...] > 0) | (rowi == coli)).astype(jnp.bfloat16)


def _zp_build(zc, bm=512):
    return pl.pallas_call(
        _zp_kernel,
        grid=(N // bm,),
        in_specs=[pl.BlockSpec((bm, N), lambda i: (i, 0))],
        out_specs=pl.BlockSpec((bm, N), lambda i: (i, 0)),
        out_shape=jax.ShapeDtypeStruct((N, N), jnp.bfloat16),
    )(zc)


def _upat_kernel(a_ref, b_ref, mb_ref, o_ref, acc_ref):
    k = pl.program_id(2)

    @pl.when(k == 0)
    def _():
        acc_ref[...] = jnp.zeros_like(acc_ref)

    bblk = b_ref[...] * mb_ref[0, :][None, :]
    acc_ref[...] += jnp.dot(a_ref[...], bblk,
                            preferred_element_type=jnp.float32)

    @pl.when(k == pl.num_programs(2) - 1)
    def _():
        o_ref[...] = (acc_ref[...] > 0).astype(jnp.bfloat16)


def _upat(zp, mb, bm=1024, bn=2048, bk=512):
    return pl.pallas_call(
        _upat_kernel,
        grid=(N // bm, N // bn, N // bk),
        in_specs=[pl.BlockSpec((bm, bk), lambda i, j, k: (i, k)),
                  pl.BlockSpec((bk, bn), lambda i, j, k: (k, j)),
                  pl.BlockSpec((1, bn), lambda i, j, k: (0, j))],
        out_specs=pl.BlockSpec((bm, bn), lambda i, j, k: (i, j)),
        out_shape=jax.ShapeDtypeStruct((N, N), jnp.bfloat16),
        scratch_shapes=[pltpu.VMEM((bm, bn), jnp.float32)],
    )(zp, zp, mb)


def _vpat_kernel(a_ref, b_ref, ma_ref, vp_ref, deg_ref, acc_ref):
    j, i, k = pl.program_id(0), pl.program_id(1), pl.program_id(2)

    @pl.when(k == 0)
    def _():
        acc_ref[...] = jnp.zeros_like(acc_ref)

    ablk = a_ref[...] * ma_ref[0, :][None, :]
    acc_ref[...] += jax.lax.dot_general(
        ablk, b_ref[...], (((0,), (0,)), ((), ())),
        preferred_element_type=jnp.float32)

    @pl.when(k == pl.num_programs(2) - 1)
    def _():
        bm, bn = acc_ref.shape
        rowi = i * bm + jax.lax.broadcasted_iota(jnp.int32, (bm, bn), 0)
        coli = j * bn + jax.lax.broadcasted_iota(jnp.int32, (bm, bn), 1)
        vp = jnp.where((acc_ref[...] > 0) & (rowi != coli), 1.0, 0.0)
        vp_ref[...] = vp.astype(jnp.bfloat16)

        @pl.when(i == 0)
        def _():
            deg_ref[...] = jnp.zeros_like(deg_ref)

        deg_ref[...] += jnp.sum(vp, axis=0)[None, :]


def _vpat(zp, mb, upat, bm=1024, bn=2048, bk=512):
    return pl.pallas_call(
        _vpat_kernel,
        grid=(N // bn, N // bm, N // bk),
        in_specs=[
            pl.BlockSpec((bk, bm), lambda j, i, k: (k, i)),
            pl.BlockSpec((bk, bn), lambda j, i, k: (k, j)),
            pl.BlockSpec((1, bm), lambda j, i, k: (0, i)),
        ],
        out_specs=[
            pl.BlockSpec((bm, bn), lambda j, i, k: (i, j)),
            pl.BlockSpec((1, bn), lambda j, i, k: (0, j)),
        ],
        out_shape=[jax.ShapeDtypeStruct((N, N), jnp.bfloat16),
                   jax.ShapeDtypeStruct((1, N), jnp.float32)],
        scratch_shapes=[pltpu.VMEM((bm, bn), jnp.float32)],
    )(zp, upat, mb)


def _dinv2_of(deg_row, m_row):
    return jnp.where(m_row > 0,
                     jax.lax.rsqrt(jnp.maximum(deg_row + m_row, 1.0)), 0.0)


def _final_kernel(vp_ref, deg_ref, m_ref, degb_ref, mb_ref, fitb_ref,
                  xnew_ref, w2_ref, b2_ref, o_ref, y_ref):
    i = pl.program_id(0)

    dinv2_all = _dinv2_of(deg_ref[0, :], m_ref[0, :])
    dinv2_blk = _dinv2_of(degb_ref[0, :], mb_ref[0, :])
    r_blk = jnp.sum(vp_ref[...].astype(jnp.float32) * dinv2_all[None, :],
                    axis=1) + mb_ref[0, :] * dinv2_blk
    coeff = (fitb_ref[0, :] * dinv2_blk * r_blk * (1.0 / K))[None, :]

    @pl.when(i == 0)
    def _():
        y_ref[...] = jnp.zeros_like(y_ref)
        o_ref[...] = jnp.zeros_like(o_ref)

    y_ref[...] += jnp.dot(coeff, xnew_ref[...],
                          preferred_element_type=jnp.float32)

    @pl.when(i == pl.num_programs(0) - 1)
    def _():
        o_ref[...] = jnp.dot(y_ref[...], w2_ref[...],
                             preferred_element_type=jnp.float32) \
            + b2_ref[0, :][None, :]


def _final(vp, deg_cols, m, fitness, x_new, W2, b2, bm=512):
    out, _ = pl.pallas_call(
        _final_kernel,
        grid=(N // bm,),
        in_specs=[
            pl.BlockSpec((bm, N), lambda i: (i, 0)),
            pl.BlockSpec((1, N), lambda i: (0, 0)),
            pl.BlockSpec((1, N), lambda i: (0, 0)),
            pl.BlockSpec((1, bm), lambda i: (0, i)),
            pl.BlockSpec((1, bm), lambda i: (0, i)),
            pl.BlockSpec((1, bm), lambda i: (0, i)),
            pl.BlockSpec((bm, D), lambda i: (i, 0)),
            pl.BlockSpec((D, D), lambda i: (0, 0)),
            pl.BlockSpec((1, D), lambda i: (0, 0)),
        ],
        out_specs=[pl.BlockSpec((1, D), lambda i: (0, 0)),
                   pl.BlockSpec((1, D), lambda i: (0, 0))],
        out_shape=[jax.ShapeDtypeStruct((1, D), jnp.float32),
                   jax.ShapeDtypeStruct((1, D), jnp.float32)],
    )(vp, deg_cols, m, deg_cols, m, fitness, x_new, W2, b2)
    return out


def _mm_kernel(a_ref, b_ref, o_ref):
    o_ref[...] = jnp.dot(a_ref[...], b_ref[...],
                         preferred_element_type=jnp.float32)


def _matmul(a, b):
    return pl.pallas_call(
        _mm_kernel,
        out_shape=jax.ShapeDtypeStruct((a.shape[0], b.shape[1]), jnp.float32),
    )(a, b)


def _mm_acc_kernel(a_ref, b_ref, o_ref):
    @pl.when(pl.program_id(2) == 0)
    def _():
        o_ref[...] = jnp.zeros_like(o_ref)

    o_ref[...] += jnp.dot(a_ref[...], b_ref[...],
                          preferred_element_type=jnp.float32)


def _matmul_big(a, b, bm=512, bn=1024, bk=1024):
    M, Kd = a.shape
    _, Nd = b.shape
    return pl.pallas_call(
        _mm_acc_kernel,
        grid=(M // bm, Nd // bn, Kd // bk),
        in_specs=[pl.BlockSpec((bm, bk), lambda i, j, k: (i, k)),
                  pl.BlockSpec((bk, bn), lambda i, j, k: (k, j))],
        out_specs=pl.BlockSpec((bm, bn), lambda i, j, k: (i, j)),
        out_shape=jax.ShapeDtypeStruct((M, Nd), jnp.float32),
    )(a, b)


def _mm_at_acc_kernel(a_ref, b_ref, o_ref):
    @pl.when(pl.program_id(2) == 0)
    def _():
        o_ref[...] = jnp.zeros_like(o_ref)

    o_ref[...] += jax.lax.dot_general(
        a_ref[...], b_ref[...], (((0,), (0,)), ((), ())),
        preferred_element_type=jnp.float32)


def _matmul_at(a, b, bm=512, bn=1024, bk=1024):
    Kd, M = a.shape
    _, Nd = b.shape
    return pl.pallas_call(
        _mm_at_acc_kernel,
        grid=(M // bm, Nd // bn, Kd // bk),
        in_specs=[pl.BlockSpec((bk, bm), lambda i, j, k: (k, i)),
                  pl.BlockSpec((bk, bn), lambda i, j, k: (k, j))],
        out_specs=pl.BlockSpec((bm, bn), lambda i, j, k: (i, j)),
        out_shape=jax.ShapeDtypeStruct((M, Nd), jnp.float32),
    )(a, b)


def kernel(x, edge_index, batch, W1, b1, lin_W, lin_b, att_W, att_b,
           le1_W, le1_b, le2_W, le3_W, le3_b, W2, b2):
    row, col = edge_index[0], edge_index[1]
    f32 = jnp.float32

    ones = jnp.ones((E,), f32)
    deg = jax.ops.segment_sum(ones, col, num_segments=N) + 1.0
    dinv = 1.0 / jnp.sqrt(deg)
    xw1 = _matmul(x, W1)
    Y = dinv[:, None] * xw1
    agg1 = jax.ops.segment_sum(Y[row], col, num_segments=N) + Y
    h = jax.nn.relu(dinv[:, None] * agg1 + b1)

    x_q = jnp.maximum(jax.ops.segment_max(h[row], col, num_segments=N), h)
    u = att_W[0, :D]
    v = att_W[0, D:]
    g = lin_W.T @ u
    c0 = jnp.dot(u, lin_b) + att_b[0]
    q = x_q @ g + c0
    p = h @ v
    pm = jnp.max(p)
    shift = jax.nn.leaky_relu(q + pm, 0.2)

    s_e = jax.nn.leaky_relu(q[col] + p[row], 0.2)
    sexp_e = jnp.exp(s_e - shift[col])
    sexp_self = jnp.exp(jax.nn.leaky_relu(q + p, 0.2) - shift)
    ssum = jax.ops.segment_sum(sexp_e, col, num_segments=N) + sexp_self
    den = ssum + 1e-16
    xnew_num = (jax.ops.segment_sum(sexp_e[:, None] * h[row], col,
                                    num_segments=N)
                + sexp_self[:, None] * h)
    x_new = xnew_num / den[:, None]

    a_s = x_new @ le1_W[0] + le1_b[0]
    bb_s = x_new @ le2_W[0]
    t_s = x_new @ le3_W[0] + le3_b[0]
    asum = jax.ops.segment_sum(a_s[row], col, num_segments=N) + a_s
    fitness = jax.nn.sigmoid(asum - deg * bb_s + t_s)

    _, perm = jax.lax.top_k(fitness, K)
    m = jnp.zeros((N,), f32).at[perm].set(1.0)

    Zc = jnp.zeros((N, N), f32).at[row, col].add(1.0)
    Zp = _zp_build(Zc)
    mbf = m.astype(jnp.bfloat16)[None, :]
    Up = _upat(Zp, mbf)
    Vp, deg_cols = _vpat(Zp, mbf, Up)

    return _final(Vp, deg_cols, m[None, :], fitness[None, :], x_new, W2,
                  b2[None, :])


# speedup vs baseline: 4.3452x; 3.1309x over previous
import functools

import jax
import jax.numpy as jnp
import numpy as np
from jax import lax
from jax.experimental import pallas as pl
from jax.experimental.pallas import tpu as pltpu
from jax.experimental.pallas import tpu_sc as plsc

N = 4096
E = 65536
K = 2048
D = 128

_NC = 2
_NS = 16
_NW = _NC * _NS
_CH = 128

_MESH = plsc.VectorSubcoreMesh(core_axis_name="c", subcore_axis_name="s",
                               num_cores=_NC, num_subcores=_NS)


def _sc_segsum(table, idxr2d, idxc2d, nacc, width, adjust=False,
               q_tab=None, p_tab=None):
    nchunks = E // _CH
    chunks_w = nchunks // _NW
    rows_w = nacc // _NS

    def body(table_ref, idxr_ref, idxc_ref, zeros_ref, q_ref, p_ref, out_ref,
             idxr_v, idxc_v, row_buf, q_v, p_v, acc_sh, sem):
        cid = lax.axis_index("c")
        sid = lax.axis_index("s")
        wid = sid * _NC + cid
        pltpu.sync_copy(zeros_ref.at[pl.ds(sid * rows_w, rows_w)],
                        acc_sh.at[pl.ds(sid * rows_w, rows_w)])
        base = wid * chunks_w
        pltpu.sync_copy(idxr_ref.at[pl.ds(base, chunks_w)], idxr_v)
        pltpu.sync_copy(idxc_ref.at[pl.ds(base, chunks_w)], idxc_v)
        if adjust:
            pltpu.sync_copy(q_ref, q_v)
            pltpu.sync_copy(p_ref, p_v)
        plsc.subcore_barrier()

        def chunk(j, carry):
            if adjust:
                def lane(l, c2):
                    rv = idxr_v[j, pl.ds(l * 16, 16)]
                    cv = idxc_v[j, pl.ds(l * 16, 16)]
                    qv = plsc.load_gather(q_v, [cv])
                    pv = plsc.load_gather(p_v, [rv])
                    off = jnp.where(qv + pv < 0.0,
                                    jnp.full((16,), nacc // 2, jnp.int32),
                                    jnp.zeros((16,), jnp.int32))
                    idxr_v[j, pl.ds(l * 16, 16)] = rv + off
                    idxc_v[j, pl.ds(l * 16, 16)] = cv + off
                    return c2
                lax.fori_loop(0, _CH // 16, lane, 0)
            pltpu.async_copy(table_ref.at[idxr_v.at[j]], row_buf, sem).wait()
            pltpu.sync_copy(row_buf, acc_sh.at[idxc_v.at[j]], add=True)
            return carry
        lax.fori_loop(0, chunks_w, chunk, 0)
        plsc.subcore_barrier()
        pltpu.sync_copy(acc_sh.at[pl.ds(sid * rows_w, rows_w)],
                        out_ref.at[pl.ds(cid * nacc + sid * rows_w, rows_w)])

    tdim = 2 if adjust else 1
    k = pl.kernel(
        body,
        out_type=jax.ShapeDtypeStruct((2 * nacc, width), jnp.float32),
        mesh=_MESH,
        compiler_params=pltpu.CompilerParams(needs_layout_passes=False),
        scratch_types=[
            pltpu.VMEM((chunks_w, _CH), jnp.int32),
            pltpu.VMEM((chunks_w, _CH), jnp.int32),
            pltpu.VMEM((_CH, width), jnp.float32),
            pltpu.VMEM((N,), jnp.float32),
            pltpu.VMEM((N,), jnp.float32),
            pltpu.VMEM_SHARED((nacc, width), jnp.float32),
            pltpu.SemaphoreType.DMA,
        ],
    )
    zeros = jnp.zeros((nacc, width), jnp.float32)
    if not adjust:
        q_tab = jnp.zeros((N,), jnp.float32)
        p_tab = q_tab
    return k(table, idxr2d, idxc2d, zeros, q_tab, p_tab)


def _zp_kernel(zc_ref, zp_ref):
    i = pl.program_id(0)
    bm = zp_ref.shape[0]
    rowi = i * bm + jax.lax.broadcasted_iota(jnp.int32, zp_ref.shape, 0)
    coli = jax.lax.broadcasted_iota(jnp.int32, zp_ref.shape, 1)
    zp_ref[...] = ((zc_ref[...] > 0) | (rowi == coli)).astype(jnp.bfloat16)


def _zp_build(zc, bm=512):
    return pl.pallas_call(
        _zp_kernel,
        grid=(N // bm,),
        in_specs=[pl.BlockSpec((bm, N), lambda i: (i, 0))],
        out_specs=pl.BlockSpec((bm, N), lambda i: (i, 0)),
        out_shape=jax.ShapeDtypeStruct((N, N), jnp.bfloat16),
    )(zc)


def _upat_kernel(a_ref, b_ref, mb_ref, o_ref, acc_ref):
    k = pl.program_id(2)

    @pl.when(k == 0)
    def _():
        acc_ref[...] = jnp.zeros_like(acc_ref)

    bblk = b_ref[...] * mb_ref[0, :][None, :]
    acc_ref[...] += jnp.dot(a_ref[...], bblk,
                            preferred_element_type=jnp.float32)

    @pl.when(k == pl.num_programs(2) - 1)
    def _():
        o_ref[...] = (acc_ref[...] > 0).astype(jnp.bfloat16)


def _upat(zp, mb, bm=1024, bn=2048, bk=512):
    return pl.pallas_call(
        _upat_kernel,
        grid=(N // bm, N // bn, N // bk),
        in_specs=[pl.BlockSpec((bm, bk), lambda i, j, k: (i, k)),
                  pl.BlockSpec((bk, bn), lambda i, j, k: (k, j)),
                  pl.BlockSpec((1, bn), lambda i, j, k: (0, j))],
        out_specs=pl.BlockSpec((bm, bn), lambda i, j, k: (i, j)),
        out_shape=jax.ShapeDtypeStruct((N, N), jnp.bfloat16),
        scratch_shapes=[pltpu.VMEM((bm, bn), jnp.float32)],
    )(zp, zp, mb)


def _vpat_kernel(a_ref, b_ref, ma_ref, vp_ref, deg_ref, acc_ref):
    j, i, k = pl.program_id(0), pl.program_id(1), pl.program_id(2)

    @pl.when(k == 0)
    def _():
        acc_ref[...] = jnp.zeros_like(acc_ref)

    ablk = a_ref[...] * ma_ref[0, :][None, :]
    acc_ref[...] += jax.lax.dot_general(
        ablk, b_ref[...], (((0,), (0,)), ((), ())),
        preferred_element_type=jnp.float32)

    @pl.when(k == pl.num_programs(2) - 1)
    def _():
        bm, bn = acc_ref.shape
        rowi = i * bm + jax.lax.broadcasted_iota(jnp.int32, (bm, bn), 0)
        coli = j * bn + jax.lax.broadcasted_iota(jnp.int32, (bm, bn), 1)
        vp = jnp.where((acc_ref[...] > 0) & (rowi != coli), 1.0, 0.0)
        vp_ref[...] = vp.astype(jnp.bfloat16)

        @pl.when(i == 0)
        def _():
            deg_ref[...] = jnp.zeros_like(deg_ref)

        deg_ref[...] += jnp.sum(vp, axis=0)[None, :]


def _vpat(zp, mb, upat, bm=1024, bn=2048, bk=512):
    return pl.pallas_call(
        _vpat_kernel,
        grid=(N // bn, N // bm, N // bk),
        in_specs=[
            pl.BlockSpec((bk, bm), lambda j, i, k: (k, i)),
            pl.BlockSpec((bk, bn), lambda j, i, k: (k, j)),
            pl.BlockSpec((1, bm), lambda j, i, k: (0, i)),
        ],
        out_specs=[
            pl.BlockSpec((bm, bn), lambda j, i, k: (i, j)),
            pl.BlockSpec((1, bn), lambda j, i, k: (0, j)),
        ],
        out_shape=[jax.ShapeDtypeStruct((N, N), jnp.bfloat16),
                   jax.ShapeDtypeStruct((1, N), jnp.float32)],
        scratch_shapes=[pltpu.VMEM((bm, bn), jnp.float32)],
    )(zp, upat, mb)


def _dinv2_of(deg_row, m_row):
    return jnp.where(m_row > 0,
                     jax.lax.rsqrt(jnp.maximum(deg_row + m_row, 1.0)), 0.0)


def _final_kernel(vp_ref, deg_ref, m_ref, degb_ref, mb_ref, fitb_ref,
                  xnew_ref, w2_ref, b2_ref, o_ref, y_ref):
    i = pl.program_id(0)

    dinv2_all = _dinv2_of(deg_ref[0, :], m_ref[0, :])
    dinv2_blk = _dinv2_of(degb_ref[0, :], mb_ref[0, :])
    r_blk = jnp.sum(vp_ref[...].astype(jnp.float32) * dinv2_all[None, :],
                    axis=1) + mb_ref[0, :] * dinv2_blk
    coeff = (fitb_ref[0, :] * dinv2_blk * r_blk * (1.0 / K))[None, :]

    @pl.when(i == 0)
    def _():
        y_ref[...] = jnp.zeros_like(y_ref)
        o_ref[...] = jnp.zeros_like(o_ref)

    y_ref[...] += jnp.dot(coeff, xnew_ref[...],
                          preferred_element_type=jnp.float32)

    @pl.when(i == pl.num_programs(0) - 1)
    def _():
        o_ref[...] = jnp.dot(y_ref[...], w2_ref[...],
                             preferred_element_type=jnp.float32) \
            + b2_ref[0, :][None, :]


def _final(vp, deg_cols, m, fitness, x_new, W2, b2, bm=512):
    out, _ = pl.pallas_call(
        _final_kernel,
        grid=(N // bm,),
        in_specs=[
            pl.BlockSpec((bm, N), lambda i: (i, 0)),
            pl.BlockSpec((1, N), lambda i: (0, 0)),
            pl.BlockSpec((1, N), lambda i: (0, 0)),
            pl.BlockSpec((1, bm), lambda i: (0, i)),
            pl.BlockSpec((1, bm), lambda i: (0, i)),
            pl.BlockSpec((1, bm), lambda i: (0, i)),
            pl.BlockSpec((bm, D), lambda i: (i, 0)),
            pl.BlockSpec((D, D), lambda i: (0, 0)),
            pl.BlockSpec((1, D), lambda i: (0, 0)),
        ],
        out_specs=[pl.BlockSpec((1, D), lambda i: (0, 0)),
                   pl.BlockSpec((1, D), lambda i: (0, 0))],
        out_shape=[jax.ShapeDtypeStruct((1, D), jnp.float32),
                   jax.ShapeDtypeStruct((1, D), jnp.float32)],
    )(vp, deg_cols, m, deg_cols, m, fitness, x_new, W2, b2)
    return out


def _mm_kernel(a_ref, b_ref, o_ref):
    o_ref[...] = jnp.dot(a_ref[...], b_ref[...],
                         preferred_element_type=jnp.float32)


def _matmul(a, b):
    return pl.pallas_call(
        _mm_kernel,
        out_shape=jax.ShapeDtypeStruct((a.shape[0], b.shape[1]), jnp.float32),
    )(a, b)


def _mm_acc_kernel(a_ref, b_ref, o_ref):
    @pl.when(pl.program_id(2) == 0)
    def _():
        o_ref[...] = jnp.zeros_like(o_ref)

    o_ref[...] += jnp.dot(a_ref[...], b_ref[...],
                          preferred_element_type=jnp.float32)


def _matmul_big(a, b, bm=512, bn=1024, bk=1024):
    M, Kd = a.shape
    _, Nd = b.shape
    return pl.pallas_call(
        _mm_acc_kernel,
        grid=(M // bm, Nd // bn, Kd // bk),
        in_specs=[pl.BlockSpec((bm, bk), lambda i, j, k: (i, k)),
                  pl.BlockSpec((bk, bn), lambda i, j, k: (k, j))],
        out_specs=pl.BlockSpec((bm, bn), lambda i, j, k: (i, j)),
        out_shape=jax.ShapeDtypeStruct((M, Nd), jnp.float32),
    )(a, b)


def _mm_at_acc_kernel(a_ref, b_ref, o_ref):
    @pl.when(pl.program_id(2) == 0)
    def _():
        o_ref[...] = jnp.zeros_like(o_ref)

    o_ref[...] += jax.lax.dot_general(
        a_ref[...], b_ref[...], (((0,), (0,)), ((), ())),
        preferred_element_type=jnp.float32)


def _matmul_at(a, b, bm=512, bn=1024, bk=1024):
    Kd, M = a.shape
    _, Nd = b.shape
    return pl.pallas_call(
        _mm_at_acc_kernel,
        grid=(M // bm, Nd // bn, Kd // bk),
        in_specs=[pl.BlockSpec((bk, bm), lambda i, j, k: (k, i)),
                  pl.BlockSpec((bk, bn), lambda i, j, k: (k, j))],
        out_specs=pl.BlockSpec((bm, bn), lambda i, j, k: (i, j)),
        out_shape=jax.ShapeDtypeStruct((M, Nd), jnp.float32),
    )(a, b)


def kernel(x, edge_index, batch, W1, b1, lin_W, lin_b, att_W, att_b,
           le1_W, le1_b, le2_W, le3_W, le3_b, W2, b2):
    row = edge_index[0].astype(jnp.int32)
    col = edge_index[1].astype(jnp.int32)
    f32 = jnp.float32
    row2d = row.reshape(E // _CH, _CH)
    col2d = col.reshape(E // _CH, _CH)

    ones_tab = jnp.zeros((N, 128), f32).at[:, 0].set(1.0)
    dega = _sc_segsum(ones_tab, col2d, col2d, N, 128)
    deg = dega[:N, 0] + dega[N:, 0] + 1.0
    dinv = jax.lax.rsqrt(deg)
    xw1 = _matmul(x, W1)
    Y = dinv[:, None] * xw1
    ab = _sc_segsum(Y, row2d, col2d, N, D)
    agg1 = ab[:N] + ab[N:] + Y
    h = jax.nn.relu(dinv[:, None] * agg1 + b1)

    x_q = jnp.maximum(jax.ops.segment_max(h[row], col, num_segments=N), h)
    u = att_W[0, :D]
    v = att_W[0, D:]
    g = lin_W.T @ u
    c0 = jnp.dot(u, lin_b) + att_b[0]
    q = x_q @ g + c0
    p = h @ v
    pm = jnp.max(p)
    shift = jax.nn.leaky_relu(q + pm, 0.2)

    ep = jnp.exp(p)
    em = jnp.exp(0.2 * p)
    hpm = jnp.concatenate([h * ep[:, None], h * em[:, None]], axis=0)
    dd = _sc_segsum(hpm, row2d, col2d, 2 * N, 128, adjust=True,
                    q_tab=q, p_tab=p)
    acc = dd[:2 * N] + dd[2 * N:]
    epm = jnp.concatenate([ep[:, None], em[:, None]], axis=0)
    epm = jnp.concatenate([epm, jnp.zeros((2 * N, 127), f32)], axis=1)
    d2 = _sc_segsum(epm, row2d, col2d, 2 * N, 128, adjust=True,
                    q_tab=q, p_tab=p)
    acc2 = d2[:2 * N, 0] + d2[2 * N:, 0]
    eqp = jnp.exp(q - shift)
    eqm = jnp.exp(0.2 * q - shift)
    sexp_self = jnp.exp(jax.nn.leaky_relu(q + p, 0.2) - shift)
    ssum = eqp * acc2[:N] + eqm * acc2[N:] + sexp_self
    den = ssum + 1e-16
    xnew_num = (eqp[:, None] * acc[:N, :D] + eqm[:, None] * acc[N:, :D]
                + sexp_self[:, None] * h)
    x_new = xnew_num / den[:, None]

    a_s = x_new @ le1_W[0] + le1_b[0]
    bb_s = x_new @ le2_W[0]
    t_s = x_new @ le3_W[0] + le3_b[0]
    as_tab = jnp.concatenate([a_s[:, None], jnp.zeros((N, 127), f32)], axis=1)
    asum_a = _sc_segsum(as_tab, row2d, col2d, N, 128)
    asum = asum_a[:N, 0] + asum_a[N:, 0] + a_s
    fitness = jax.nn.sigmoid(asum - deg * bb_s + t_s)

    _, perm = jax.lax.top_k(fitness, K)
    m = jnp.zeros((N,), f32).at[perm].set(1.0)

    Zc = jnp.zeros((N, N), f32).at[row, col].add(1.0)
    Zp = _zp_build(Zc)
    mbf = m.astype(jnp.bfloat16)[None, :]
    Up = _upat(Zp, mbf)
    Vp, deg_cols = _vpat(Zp, mbf, Up)

    return _final(Vp, deg_cols, m[None, :], fitness[None, :], x_new, W2,
                  b2[None, :])


# speedup vs baseline: 4.5125x; 1.0385x over previous
import functools

import jax
import jax.numpy as jnp
import numpy as np
from jax import lax
from jax.experimental import pallas as pl
from jax.experimental.pallas import tpu as pltpu
from jax.experimental.pallas import tpu_sc as plsc

N = 4096
E = 65536
K = 2048
D = 128

_NC = 2
_NS = 16
_NW = _NC * _NS
_CH = 128

_MESH = plsc.VectorSubcoreMesh(core_axis_name="c", subcore_axis_name="s",
                               num_cores=_NC, num_subcores=_NS)


def _sc_segsum(table, idxr2d, idxc2d, nacc, width, adjust=False,
               q_tab=None, p_tab=None, gather=True):
    nchunks = E // _CH
    chunks_w = nchunks // _NW
    rows_w = nacc // _NS

    def body(table_ref, idxr_ref, idxc_ref, zeros_ref, q_ref, p_ref, out_ref,
             idxr_v, idxc_v, row_buf, row_buf2, q_v, p_v, acc_sh, sem):
        cid = lax.axis_index("c")
        sid = lax.axis_index("s")
        wid = sid * _NC + cid
        pltpu.sync_copy(zeros_ref.at[pl.ds(sid * rows_w, rows_w)],
                        acc_sh.at[pl.ds(sid * rows_w, rows_w)])
        base = wid * chunks_w
        if gather:
            pltpu.sync_copy(idxr_ref.at[pl.ds(base, chunks_w)], idxr_v)
        pltpu.sync_copy(idxc_ref.at[pl.ds(base, chunks_w)], idxc_v)
        if adjust:
            pltpu.sync_copy(q_ref, q_v)
            pltpu.sync_copy(p_ref, p_v)
        plsc.subcore_barrier()

        def adjust_chunk(j):
            for l in range(_CH // 16):
                rv = idxr_v[j, pl.ds(l * 16, 16)]
                cv = idxc_v[j, pl.ds(l * 16, 16)]
                qv = plsc.load_gather(q_v, [cv])
                pv = plsc.load_gather(p_v, [rv])
                off = jnp.where(qv + pv < 0.0,
                                jnp.full((16,), nacc // 2, jnp.int32),
                                jnp.zeros((16,), jnp.int32))
                idxr_v[j, pl.ds(l * 16, 16)] = rv + off
                idxc_v[j, pl.ds(l * 16, 16)] = cv + off

        if gather:
            if adjust:
                def adj(j, c):
                    adjust_chunk(j)
                    return c
                lax.fori_loop(0, chunks_w, adj, 0)
            bufs = (row_buf, row_buf2)
            pltpu.async_copy(table_ref.at[idxr_v.at[0]], bufs[0], sem)

            def step(j, c):
                even = j % 2 == 0

                @pl.when(j + 1 < chunks_w)
                def _():
                    @pl.when(even)
                    def _():
                        pltpu.async_copy(table_ref.at[idxr_v.at[j + 1]],
                                         bufs[1], sem)

                    @pl.when(jnp.logical_not(even))
                    def _():
                        pltpu.async_copy(table_ref.at[idxr_v.at[j + 1]],
                                         bufs[0], sem)

                @pl.when(even)
                def _():
                    pltpu.make_async_copy(table_ref.at[idxr_v.at[j]],
                                          bufs[0], sem).wait()
                    pltpu.sync_copy(bufs[0], acc_sh.at[idxc_v.at[j]],
                                    add=True)

                @pl.when(jnp.logical_not(even))
                def _():
                    pltpu.make_async_copy(table_ref.at[idxr_v.at[j]],
                                          bufs[1], sem).wait()
                    pltpu.sync_copy(bufs[1], acc_sh.at[idxc_v.at[j]],
                                    add=True)
                return c
            lax.fori_loop(0, chunks_w, step, 0)
        else:
            pltpu.sync_copy(table_ref.at[pl.ds(0, _CH)], row_buf)

            def step0(j, c):
                pltpu.sync_copy(row_buf, acc_sh.at[idxc_v.at[j]], add=True)
                return c
            lax.fori_loop(0, chunks_w, step0, 0)
        plsc.subcore_barrier()
        pltpu.sync_copy(acc_sh.at[pl.ds(sid * rows_w, rows_w)],
                        out_ref.at[pl.ds(cid * nacc + sid * rows_w, rows_w)])

    tdim = 2 if adjust else 1
    k = pl.kernel(
        body,
        out_type=jax.ShapeDtypeStruct((2 * nacc, width), jnp.float32),
        mesh=_MESH,
        compiler_params=pltpu.CompilerParams(needs_layout_passes=False),
        scratch_types=[
            pltpu.VMEM((chunks_w, _CH), jnp.int32),
            pltpu.VMEM((chunks_w, _CH), jnp.int32),
            pltpu.VMEM((_CH, width), jnp.float32),
            pltpu.VMEM((_CH, width), jnp.float32),
            pltpu.VMEM((N,), jnp.float32),
            pltpu.VMEM((N,), jnp.float32),
            pltpu.VMEM_SHARED((nacc, width), jnp.float32),
            pltpu.SemaphoreType.DMA,
        ],
    )
    zeros = jnp.zeros((nacc, width), jnp.float32)
    if not adjust:
        q_tab = jnp.zeros((N,), jnp.float32)
        p_tab = q_tab
    return k(table, idxr2d, idxc2d, zeros, q_tab, p_tab)


def _zp_kernel(zc_ref, zp_ref):
    i = pl.program_id(0)
    bm = zp_ref.shape[0]
    rowi = i * bm + jax.lax.broadcasted_iota(jnp.int32, zp_ref.shape, 0)
    coli = jax.lax.broadcasted_iota(jnp.int32, zp_ref.shape, 1)
    zp_ref[...] = ((zc_ref[...] > 0) | (rowi == coli)).astype(jnp.bfloat16)


def _zp_build(zc, bm=512):
    return pl.pallas_call(
        _zp_kernel,
        grid=(N // bm,),
        in_specs=[pl.BlockSpec((bm, N), lambda i: (i, 0))],
        out_specs=pl.BlockSpec((bm, N), lambda i: (i, 0)),
        out_shape=jax.ShapeDtypeStruct((N, N), jnp.bfloat16),
    )(zc)


def _upat_kernel(a_ref, b_ref, mb_ref, o_ref, acc_ref):
    k = pl.program_id(2)

    @pl.when(k == 0)
    def _():
        acc_ref[...] = jnp.zeros_like(acc_ref)

    bblk = b_ref[...] * mb_ref[0, :][None, :]
    acc_ref[...] += jnp.dot(a_ref[...], bblk,
                            preferred_element_type=jnp.float32)

    @pl.when(k == pl.num_programs(2) - 1)
    def _():
        o_ref[...] = (acc_ref[...] > 0).astype(jnp.bfloat16)


def _upat(zp, mb, bm=1024, bn=2048, bk=512):
    return pl.pallas_call(
        _upat_kernel,
        grid=(N // bm, N // bn, N // bk),
        in_specs=[pl.BlockSpec((bm, bk), lambda i, j, k: (i, k)),
                  pl.BlockSpec((bk, bn), lambda i, j, k: (k, j)),
                  pl.BlockSpec((1, bn), lambda i, j, k: (0, j))],
        out_specs=pl.BlockSpec((bm, bn), lambda i, j, k: (i, j)),
        out_shape=jax.ShapeDtypeStruct((N, N), jnp.bfloat16),
        scratch_shapes=[pltpu.VMEM((bm, bn), jnp.float32)],
    )(zp, zp, mb)


def _vpat_kernel(a_ref, b_ref, ma_ref, vp_ref, deg_ref, acc_ref):
    j, i, k = pl.program_id(0), pl.program_id(1), pl.program_id(2)

    @pl.when(k == 0)
    def _():
        acc_ref[...] = jnp.zeros_like(acc_ref)

    ablk = a_ref[...] * ma_ref[0, :][None, :]
    acc_ref[...] += jax.lax.dot_general(
        ablk, b_ref[...], (((0,), (0,)), ((), ())),
        preferred_element_type=jnp.float32)

    @pl.when(k == pl.num_programs(2) - 1)
    def _():
        bm, bn = acc_ref.shape
        rowi = i * bm + jax.lax.broadcasted_iota(jnp.int32, (bm, bn), 0)
        coli = j * bn + jax.lax.broadcasted_iota(jnp.int32, (bm, bn), 1)
        vp = jnp.where((acc_ref[...] > 0) & (rowi != coli), 1.0, 0.0)
        vp_ref[...] = vp.astype(jnp.bfloat16)

        @pl.when(i == 0)
        def _():
            deg_ref[...] = jnp.zeros_like(deg_ref)

        deg_ref[...] += jnp.sum(vp, axis=0)[None, :]


def _vpat(zp, mb, upat, bm=1024, bn=2048, bk=512):
    return pl.pallas_call(
        _vpat_kernel,
        grid=(N // bn, N // bm, N // bk),
        in_specs=[
            pl.BlockSpec((bk, bm), lambda j, i, k: (k, i)),
            pl.BlockSpec((bk, bn), lambda j, i, k: (k, j)),
            pl.BlockSpec((1, bm), lambda j, i, k: (0, i)),
        ],
        out_specs=[
            pl.BlockSpec((bm, bn), lambda j, i, k: (i, j)),
            pl.BlockSpec((1, bn), lambda j, i, k: (0, j)),
        ],
        out_shape=[jax.ShapeDtypeStruct((N, N), jnp.bfloat16),
                   jax.ShapeDtypeStruct((1, N), jnp.float32)],
        scratch_shapes=[pltpu.VMEM((bm, bn), jnp.float32)],
    )(zp, upat, mb)


def _dinv2_of(deg_row, m_row):
    return jnp.where(m_row > 0,
                     jax.lax.rsqrt(jnp.maximum(deg_row + m_row, 1.0)), 0.0)


def _final_kernel(vp_ref, deg_ref, m_ref, degb_ref, mb_ref, fitb_ref,
                  xnew_ref, w2_ref, b2_ref, o_ref, y_ref):
    i = pl.program_id(0)

    dinv2_all = _dinv2_of(deg_ref[0, :], m_ref[0, :])
    dinv2_blk = _dinv2_of(degb_ref[0, :], mb_ref[0, :])
    r_blk = jnp.sum(vp_ref[...].astype(jnp.float32) * dinv2_all[None, :],
                    axis=1) + mb_ref[0, :] * dinv2_blk
    coeff = (fitb_ref[0, :] * dinv2_blk * r_blk * (1.0 / K))[None, :]

    @pl.when(i == 0)
    def _():
        y_ref[...] = jnp.zeros_like(y_ref)
        o_ref[...] = jnp.zeros_like(o_ref)

    y_ref[...] += jnp.dot(coeff, xnew_ref[...],
                          preferred_element_type=jnp.float32)

    @pl.when(i == pl.num_programs(0) - 1)
    def _():
        o_ref[...] = jnp.dot(y_ref[...], w2_ref[...],
                             preferred_element_type=jnp.float32) \
            + b2_ref[0, :][None, :]


def _final(vp, deg_cols, m, fitness, x_new, W2, b2, bm=512):
    out, _ = pl.pallas_call(
        _final_kernel,
        grid=(N // bm,),
        in_specs=[
            pl.BlockSpec((bm, N), lambda i: (i, 0)),
            pl.BlockSpec((1, N), lambda i: (0, 0)),
            pl.BlockSpec((1, N), lambda i: (0, 0)),
            pl.BlockSpec((1, bm), lambda i: (0, i)),
            pl.BlockSpec((1, bm), lambda i: (0, i)),
            pl.BlockSpec((1, bm), lambda i: (0, i)),
            pl.BlockSpec((bm, D), lambda i: (i, 0)),
            pl.BlockSpec((D, D), lambda i: (0, 0)),
            pl.BlockSpec((1, D), lambda i: (0, 0)),
        ],
        out_specs=[pl.BlockSpec((1, D), lambda i: (0, 0)),
                   pl.BlockSpec((1, D), lambda i: (0, 0))],
        out_shape=[jax.ShapeDtypeStruct((1, D), jnp.float32),
                   jax.ShapeDtypeStruct((1, D), jnp.float32)],
    )(vp, deg_cols, m, deg_cols, m, fitness, x_new, W2, b2)
    return out


def _mm_kernel(a_ref, b_ref, o_ref):
    o_ref[...] = jnp.dot(a_ref[...], b_ref[...],
                         preferred_element_type=jnp.float32)


def _matmul(a, b):
    return pl.pallas_call(
        _mm_kernel,
        out_shape=jax.ShapeDtypeStruct((a.shape[0], b.shape[1]), jnp.float32),
    )(a, b)


def _mm_acc_kernel(a_ref, b_ref, o_ref):
    @pl.when(pl.program_id(2) == 0)
    def _():
        o_ref[...] = jnp.zeros_like(o_ref)

    o_ref[...] += jnp.dot(a_ref[...], b_ref[...],
                          preferred_element_type=jnp.float32)


def _matmul_big(a, b, bm=512, bn=1024, bk=1024):
    M, Kd = a.shape
    _, Nd = b.shape
    return pl.pallas_call(
        _mm_acc_kernel,
        grid=(M // bm, Nd // bn, Kd // bk),
        in_specs=[pl.BlockSpec((bm, bk), lambda i, j, k: (i, k)),
                  pl.BlockSpec((bk, bn), lambda i, j, k: (k, j))],
        out_specs=pl.BlockSpec((bm, bn), lambda i, j, k: (i, j)),
        out_shape=jax.ShapeDtypeStruct((M, Nd), jnp.float32),
    )(a, b)


def _mm_at_acc_kernel(a_ref, b_ref, o_ref):
    @pl.when(pl.program_id(2) == 0)
    def _():
        o_ref[...] = jnp.zeros_like(o_ref)

    o_ref[...] += jax.lax.dot_general(
        a_ref[...], b_ref[...], (((0,), (0,)), ((), ())),
        preferred_element_type=jnp.float32)


def _matmul_at(a, b, bm=512, bn=1024, bk=1024):
    Kd, M = a.shape
    _, Nd = b.shape
    return pl.pallas_call(
        _mm_at_acc_kernel,
        grid=(M // bm, Nd // bn, Kd // bk),
        in_specs=[pl.BlockSpec((bk, bm), lambda i, j, k: (k, i)),
                  pl.BlockSpec((bk, bn), lambda i, j, k: (k, j))],
        out_specs=pl.BlockSpec((bm, bn), lambda i, j, k: (i, j)),
        out_shape=jax.ShapeDtypeStruct((M, Nd), jnp.float32),
    )(a, b)


def kernel(x, edge_index, batch, W1, b1, lin_W, lin_b, att_W, att_b,
           le1_W, le1_b, le2_W, le3_W, le3_b, W2, b2):
    row = edge_index[0].astype(jnp.int32)
    col = edge_index[1].astype(jnp.int32)
    f32 = jnp.float32
    row2d = row.reshape(E // _CH, _CH)
    col2d = col.reshape(E // _CH, _CH)

    ones_tab = jnp.zeros((_CH, 128), f32).at[:, 0].set(1.0)
    dega = _sc_segsum(ones_tab, col2d, col2d, N, 128, gather=False)
    deg = dega[:N, 0] + dega[N:, 0] + 1.0
    dinv = jax.lax.rsqrt(deg)
    xw1 = _matmul(x, W1)
    Y = dinv[:, None] * xw1
    ab = _sc_segsum(Y, row2d, col2d, N, D)
    agg1 = ab[:N] + ab[N:] + Y
    h = jax.nn.relu(dinv[:, None] * agg1 + b1)

    x_q = jnp.maximum(jax.ops.segment_max(h[row], col, num_segments=N), h)
    u = att_W[0, :D]
    v = att_W[0, D:]
    g = lin_W.T @ u
    c0 = jnp.dot(u, lin_b) + att_b[0]
    q = x_q @ g + c0
    p = h @ v
    pm = jnp.max(p)
    shift = jax.nn.leaky_relu(q + pm, 0.2)

    ep = jnp.exp(p)
    em = jnp.exp(0.2 * p)
    hpm = jnp.concatenate([h * ep[:, None], h * em[:, None]], axis=0)
    dd = _sc_segsum(hpm, row2d, col2d, 2 * N, 128, adjust=True,
                    q_tab=q, p_tab=p)
    acc = dd[:2 * N] + dd[2 * N:]
    epm = jnp.concatenate([ep[:, None], em[:, None]], axis=0)
    epm = jnp.concatenate([epm, jnp.zeros((2 * N, 127), f32)], axis=1)
    d2 = _sc_segsum(epm, row2d, col2d, 2 * N, 128, adjust=True,
                    q_tab=q, p_tab=p)
    acc2 = d2[:2 * N, 0] + d2[2 * N:, 0]
    eqp = jnp.exp(q - shift)
    eqm = jnp.exp(0.2 * q - shift)
    sexp_self = jnp.exp(jax.nn.leaky_relu(q + p, 0.2) - shift)
    ssum = eqp * acc2[:N] + eqm * acc2[N:] + sexp_self
    den = ssum + 1e-16
    xnew_num = (eqp[:, None] * acc[:N, :D] + eqm[:, None] * acc[N:, :D]
                + sexp_self[:, None] * h)
    x_new = xnew_num / den[:, None]

    a_s = x_new @ le1_W[0] + le1_b[0]
    bb_s = x_new @ le2_W[0]
    t_s = x_new @ le3_W[0] + le3_b[0]
    as_tab = jnp.concatenate([a_s[:, None], jnp.zeros((N, 127), f32)], axis=1)
    asum_a = _sc_segsum(as_tab, row2d, col2d, N, 128)
    asum = asum_a[:N, 0] + asum_a[N:, 0] + a_s
    fitness = jax.nn.sigmoid(asum - deg * bb_s + t_s)

    _, perm = jax.lax.top_k(fitness, K)
    m = jnp.zeros((N,), f32).at[perm].set(1.0)

    Zc = jnp.zeros((N, N), f32).at[row, col].add(1.0)
    Zp = _zp_build(Zc)
    mbf = m.astype(jnp.bfloat16)[None, :]
    Up = _upat(Zp, mbf)
    Vp, deg_cols = _vpat(Zp, mbf, Up)

    return _final(Vp, deg_cols, m[None, :], fitness[None, :], x_new, W2,
                  b2[None, :])
